# Initial kernel scaffold; baseline (speedup 1.0000x reference)
import functools

import jax
import jax.numpy as jnp
from jax import lax
from jax.experimental import pallas as pl
from jax.experimental.pallas import tpu as pltpu
from jax.experimental.pallas import tpu_sc as plsc

N = 10000
E = 320000
D = 128
H = 64
C = 40
CP = 48
DEGW = 16

NC, NS = 2, 16
NW = NC * NS

NP = N + 16
ZROWS = NP // NS
OROWS = N // NS

CHUNK = 128
NCH = (E + CHUNK - 1) // CHUNK
NCHP = ((NCH + NW - 1) // NW) * NW
CPT = NCHP // NW

_mesh = plsc.VectorSubcoreMesh(
    core_axis_name="c", subcore_axis_name="s", num_cores=NC, num_subcores=NS
)


def _fill(ref, value, nrows, width):
    vec = jnp.full((16,), value, jnp.float32)

    def row(i, carry):
        for j in range(width // 16):
            ref[i, pl.ds(j * 16, 16)] = vec
        return carry

    lax.fori_loop(0, nrows, row, 0)


def _zero_acc_slice(zbuf, acc, s):
    base = s * ZROWS
    full, rem = ZROWS // CHUNK, ZROWS % CHUNK
    for k in range(full):
        pltpu.sync_copy(zbuf, acc.at[pl.ds(base + k * CHUNK, CHUNK)])
    if rem:
        pltpu.sync_copy(
            zbuf.at[pl.ds(0, rem)], acc.at[pl.ds(base + full * CHUNK, rem)]
        )


def _copy_out(acc, out_hbm, c, s):
    base = s * OROWS
    pltpu.sync_copy(acc.at[pl.ds(base, OROWS)], out_hbm.at[c, pl.ds(base, OROWS)])


def _make_degree_kernel():
    @functools.partial(
        pl.kernel,
        out_type=jax.ShapeDtypeStruct((NC, N, DEGW), jnp.float32),
        mesh=_mesh,
        scratch_types=[
            pltpu.VMEM((CPT, CHUNK), jnp.int32),
            pltpu.VMEM((CHUNK, DEGW), jnp.float32),
            pltpu.VMEM_SHARED((NP, DEGW), jnp.float32),
        ],
    )
    def deg_kernel(dst_hbm, out_hbm, didx, vals, acc):
        c = lax.axis_index("c")
        s = lax.axis_index("s")
        wid = c * NS + s
        pltpu.sync_copy(dst_hbm.at[pl.ds(wid * CPT, CPT)], didx)
        _fill(vals, 0.0, CHUNK, DEGW)
        _zero_acc_slice(vals, acc, s)
        _fill(vals, 1.0, CHUNK, DEGW)
        plsc.subcore_barrier()

        def step(t, carry):
            pltpu.sync_copy(vals, acc.at[didx.at[t]], add=True)
            return carry

        lax.fori_loop(0, CPT, step, 0)
        plsc.subcore_barrier()
        _copy_out(acc, out_hbm, c, s)

    return deg_kernel


def _make_scatter_kernel(width):
    @functools.partial(
        pl.kernel,
        out_type=jax.ShapeDtypeStruct((NC, N, width), jnp.float32),
        mesh=_mesh,
        scratch_types=[
            pltpu.VMEM((CPT, CHUNK), jnp.int32),
            pltpu.VMEM((CPT, CHUNK), jnp.int32),
            pltpu.VMEM((CHUNK, width), jnp.float32),
            pltpu.VMEM_SHARED((NP, width), jnp.float32),
            pltpu.SemaphoreType.DMA,
        ],
    )
    def scat_kernel(g_hbm, src_hbm, dst_hbm, out_hbm, sidx, didx, rows, acc, sem):
        c = lax.axis_index("c")
        s = lax.axis_index("s")
        wid = c * NS + s
        pltpu.sync_copy(src_hbm.at[pl.ds(wid * CPT, CPT)], sidx)
        pltpu.sync_copy(dst_hbm.at[pl.ds(wid * CPT, CPT)], didx)
        _fill(rows, 0.0, CHUNK, width)
        _zero_acc_slice(rows, acc, s)
        plsc.subcore_barrier()

        def step(t, carry):
            pltpu.async_copy(g_hbm.at[sidx.at[t]], rows, sem).wait()
            pltpu.sync_copy(rows, acc.at[didx.at[t]], add=True)
            return carry

        lax.fori_loop(0, CPT, step, 0)
        plsc.subcore_barrier()
        _copy_out(acc, out_hbm, c, s)

    return scat_kernel


_degree_kernel = _make_degree_kernel()
_scatter_h = _make_scatter_kernel(H)
_scatter_c = _make_scatter_kernel(CP)

BN = 2000


def _dinv_block(da_ref, db_ref):
    deg = da_ref[:, 0:1] + db_ref[:, 0:1] + 1.0
    return lax.rsqrt(deg)


def _tc1_body(x_ref, w1_ref, da_ref, db_ref, g1_ref):
    dinv = _dinv_block(da_ref, db_ref)
    h = jnp.dot(x_ref[...], w1_ref[...], preferred_element_type=jnp.float32)
    g1_ref[...] = h * dinv


def _tc2_body(g1_ref, sa_ref, sb_ref, da_ref, db_ref, b1_ref, w2_ref, g2_ref):
    dinv = _dinv_block(da_ref, db_ref)
    h1 = dinv * (sa_ref[...] + sb_ref[...] + g1_ref[...]) + b1_ref[...]
    h1 = jnp.maximum(h1, 0.0)
    h2 = jnp.dot(h1, w2_ref[...], preferred_element_type=jnp.float32)
    g2_ref[...] = h2 * dinv


def _tc3_body(g2_ref, sa_ref, sb_ref, da_ref, db_ref, b2_ref, out_ref):
    dinv = _dinv_block(da_ref, db_ref)
    res = dinv * (sa_ref[...] + sb_ref[...] + g2_ref[...])
    out_ref[...] = res[:, :C] + b2_ref[...]


def _row_spec(w):
    return pl.BlockSpec((BN, w), lambda i: (i, 0))


def _full_spec(r, w):
    return pl.BlockSpec((r, w), lambda i: (0, 0))


@jax.jit
def kernel(x, edge_index, W1, b1, W2, b2):
    src = edge_index[0]
    dst = edge_index[1]
    pad = NCHP * CHUNK - E
    srcp = jnp.concatenate([src, jnp.zeros((pad,), jnp.int32)]).reshape(NCHP, CHUNK)
    dstp = jnp.concatenate([dst, jnp.full((pad,), N, jnp.int32)]).reshape(NCHP, CHUNK)

    deg2 = _degree_kernel(dstp)
    degA, degB = deg2[0], deg2[1]

    g1 = pl.pallas_call(
        _tc1_body,
        grid=(N // BN,),
        in_specs=[
            _row_spec(D),
            _full_spec(D, H),
            _row_spec(DEGW),
            _row_spec(DEGW),
        ],
        out_specs=_row_spec(H),
        out_shape=jax.ShapeDtypeStruct((N, H), jnp.float32),
    )(x, W1, degA, degB)

    s1 = _scatter_h(g1, srcp, dstp)

    W2p = jnp.concatenate([W2, jnp.zeros((H, CP - C), jnp.float32)], axis=1)
    g2 = pl.pallas_call(
        _tc2_body,
        grid=(N // BN,),
        in_specs=[
            _row_spec(H),
            _row_spec(H),
            _row_spec(H),
            _row_spec(DEGW),
            _row_spec(DEGW),
            _full_spec(1, H),
            _full_spec(H, CP),
        ],
        out_specs=_row_spec(CP),
        out_shape=jax.ShapeDtypeStruct((N, CP), jnp.float32),
    )(g1, s1[0], s1[1], degA, degB, b1[None, :], W2p)

    s2 = _scatter_c(g2, srcp, dstp)

    logits = pl.pallas_call(
        _tc3_body,
        grid=(N // BN,),
        in_specs=[
            _row_spec(CP),
            _row_spec(CP),
            _row_spec(CP),
            _row_spec(DEGW),
            _row_spec(DEGW),
            _full_spec(1, C),
        ],
        out_specs=_row_spec(C),
        out_shape=jax.ShapeDtypeStruct((N, C), jnp.float32),
    )(g2, s2[0], s2[1], degA, degB, b2[None, :])

    return logits


# speedup vs baseline: 14.6582x; 14.6582x over previous
import functools

import jax
import jax.numpy as jnp
from jax import lax
from jax.experimental import pallas as pl
from jax.experimental.pallas import tpu as pltpu
from jax.experimental.pallas import tpu_sc as plsc

N = 10000
E = 320000
D = 128
H = 64
C = 40
CP = 48
DEGW = 16

NC, NS = 2, 16
NW = NC * NS

NP = ((N // NS + 15) // 8) * 8 * NS
ZROWS = NP // NS

CHUNK = 128
NCH = (E + CHUNK - 1) // CHUNK
NCHP = ((NCH + 8 * NW - 1) // (8 * NW)) * 8 * NW
CPT = NCHP // NW

_mesh = plsc.VectorSubcoreMesh(
    core_axis_name="c", subcore_axis_name="s", num_cores=NC, num_subcores=NS
)
_sc_params = pltpu.CompilerParams(use_tc_tiling_on_sc=False)


def _fill(ref, value, nrows, width):
    vec = jnp.full((16,), value, jnp.float32)

    def row(i, carry):
        for j in range(width // 16):
            ref[i, pl.ds(j * 16, 16)] = vec
        return carry

    lax.fori_loop(0, nrows, row, 0)


def _zero_acc_slice(zbuf, acc, s):
    base = s * ZROWS
    full, rem = ZROWS // CHUNK, ZROWS % CHUNK
    for k in range(full):
        pltpu.sync_copy(zbuf, acc.at[pl.ds(base + k * CHUNK, CHUNK)])
    if rem:
        pltpu.sync_copy(
            zbuf.at[pl.ds(0, rem)], acc.at[pl.ds(base + full * CHUNK, rem)]
        )


def _copy_out(acc, out_hbm, c, s):
    base = s * ZROWS
    pltpu.sync_copy(acc.at[pl.ds(base, ZROWS)], out_hbm.at[c, pl.ds(base, ZROWS)])


def _make_degree_kernel():
    @functools.partial(
        pl.kernel,
        out_type=jax.ShapeDtypeStruct((NC, NP, DEGW), jnp.float32),
        mesh=_mesh,
        scratch_types=[
            pltpu.VMEM((CPT, CHUNK), jnp.int32),
            pltpu.VMEM((CHUNK, DEGW), jnp.float32),
            pltpu.VMEM_SHARED((NP, DEGW), jnp.float32),
        ],
        compiler_params=_sc_params,
    )
    def deg_kernel(dst_hbm, out_hbm, didx, vals, acc):
        c = lax.axis_index("c")
        s = lax.axis_index("s")
        wid = c * NS + s
        pltpu.sync_copy(dst_hbm.at[pl.ds(wid * CPT, CPT)], didx)
        _fill(vals, 0.0, CHUNK, DEGW)
        _zero_acc_slice(vals, acc, s)
        _fill(vals, 1.0, CHUNK, DEGW)
        plsc.subcore_barrier()

        def step(t, carry):
            pltpu.sync_copy(vals, acc.at[didx.at[t]], add=True)
            return carry

        lax.fori_loop(0, CPT, step, 0)
        plsc.subcore_barrier()
        _copy_out(acc, out_hbm, c, s)

    return deg_kernel


def _make_scatter_kernel(width):
    @functools.partial(
        pl.kernel,
        out_type=jax.ShapeDtypeStruct((NC, NP, width), jnp.float32),
        mesh=_mesh,
        scratch_types=[
            pltpu.VMEM((CPT, CHUNK), jnp.int32),
            pltpu.VMEM((CPT, CHUNK), jnp.int32),
            pltpu.VMEM((CHUNK, width), jnp.float32),
            pltpu.VMEM_SHARED((NP, width), jnp.float32),
            pltpu.SemaphoreType.DMA,
        ],
        compiler_params=_sc_params,
    )
    def scat_kernel(g_hbm, src_hbm, dst_hbm, out_hbm, sidx, didx, rows, acc, sem):
        c = lax.axis_index("c")
        s = lax.axis_index("s")
        wid = c * NS + s
        pltpu.sync_copy(src_hbm.at[pl.ds(wid * CPT, CPT)], sidx)
        pltpu.sync_copy(dst_hbm.at[pl.ds(wid * CPT, CPT)], didx)
        _fill(rows, 0.0, CHUNK, width)
        _zero_acc_slice(rows, acc, s)
        plsc.subcore_barrier()

        def step(t, carry):
            pltpu.async_copy(g_hbm.at[sidx.at[t]], rows, sem).wait()
            pltpu.sync_copy(rows, acc.at[didx.at[t]], add=True)
            return carry

        lax.fori_loop(0, CPT, step, 0)
        plsc.subcore_barrier()
        _copy_out(acc, out_hbm, c, s)

    return scat_kernel


_degree_kernel = _make_degree_kernel()
_scatter_h = _make_scatter_kernel(H)
_scatter_c = _make_scatter_kernel(CP)

BN = 2000


def _dinv_block(da_ref, db_ref):
    deg = da_ref[:, 0:1] + db_ref[:, 0:1] + 1.0
    return lax.rsqrt(deg)


def _tc1_body(x_ref, w1_ref, da_ref, db_ref, g1_ref):
    dinv = _dinv_block(da_ref, db_ref)
    h = jnp.dot(x_ref[...], w1_ref[...], preferred_element_type=jnp.float32)
    g1_ref[...] = h * dinv


def _tc2_body(g1_ref, sa_ref, sb_ref, da_ref, db_ref, b1_ref, w2_ref, g2_ref):
    dinv = _dinv_block(da_ref, db_ref)
    h1 = dinv * (sa_ref[...] + sb_ref[...] + g1_ref[...]) + b1_ref[...]
    h1 = jnp.maximum(h1, 0.0)
    h2 = jnp.dot(h1, w2_ref[...], preferred_element_type=jnp.float32)
    g2_ref[...] = h2 * dinv


def _tc3_body(g2_ref, sa_ref, sb_ref, da_ref, db_ref, b2_ref, out_ref):
    dinv = _dinv_block(da_ref, db_ref)
    res = dinv * (sa_ref[...] + sb_ref[...] + g2_ref[...])
    out_ref[...] = res[:, :C] + b2_ref[...]


def _row_spec(w):
    return pl.BlockSpec((BN, w), lambda i: (i, 0))


def _full_spec(r, w):
    return pl.BlockSpec((r, w), lambda i: (0, 0))


@jax.jit
def kernel(x, edge_index, W1, b1, W2, b2):
    src = edge_index[0]
    dst = edge_index[1]
    pad = NCHP * CHUNK - E
    srcp = jnp.concatenate([src, jnp.zeros((pad,), jnp.int32)]).reshape(NCHP, CHUNK)
    dstp = jnp.concatenate([dst, jnp.full((pad,), N, jnp.int32)]).reshape(NCHP, CHUNK)

    deg2 = _degree_kernel(dstp)
    degA, degB = deg2[0], deg2[1]

    g1 = pl.pallas_call(
        _tc1_body,
        grid=(N // BN,),
        in_specs=[
            _row_spec(D),
            _full_spec(D, H),
            _row_spec(DEGW),
            _row_spec(DEGW),
        ],
        out_specs=_row_spec(H),
        out_shape=jax.ShapeDtypeStruct((N, H), jnp.float32),
    )(x, W1, degA, degB)

    s1 = _scatter_h(g1, srcp, dstp)

    W2p = jnp.concatenate([W2, jnp.zeros((H, CP - C), jnp.float32)], axis=1)
    g2 = pl.pallas_call(
        _tc2_body,
        grid=(N // BN,),
        in_specs=[
            _row_spec(H),
            _row_spec(H),
            _row_spec(H),
            _row_spec(DEGW),
            _row_spec(DEGW),
            _full_spec(1, H),
            _full_spec(H, CP),
        ],
        out_specs=_row_spec(CP),
        out_shape=jax.ShapeDtypeStruct((N, CP), jnp.float32),
    )(g1, s1[0], s1[1], degA, degB, b1[None, :], W2p)

    s2 = _scatter_c(g2, srcp, dstp)

    logits = pl.pallas_call(
        _tc3_body,
        grid=(N // BN,),
        in_specs=[
            _row_spec(CP),
            _row_spec(CP),
            _row_spec(CP),
            _row_spec(DEGW),
            _row_spec(DEGW),
            _full_spec(1, C),
        ],
        out_specs=_row_spec(C),
        out_shape=jax.ShapeDtypeStruct((N, C), jnp.float32),
    )(g2, s2[0], s2[1], degA, degB, b2[None, :])

    return logits


# speedup vs baseline: 17.0113x; 1.1605x over previous
import functools

import jax
import jax.numpy as jnp
from jax import lax
from jax.experimental import pallas as pl
from jax.experimental.pallas import tpu as pltpu
from jax.experimental.pallas import tpu_sc as plsc

N = 10000
E = 320000
D = 128
H = 64
C = 40
CP = 48
DEGW = 16

NC, NS = 2, 16
NW = NC * NS

NP = ((N // NS + 15) // 8) * 8 * NS
ZROWS = NP // NS

CHUNK = 128
NCH = (E + CHUNK - 1) // CHUNK
NCHP = ((NCH + 8 * NW - 1) // (8 * NW)) * 8 * NW
CPT = NCHP // NW

_mesh = plsc.VectorSubcoreMesh(
    core_axis_name="c", subcore_axis_name="s", num_cores=NC, num_subcores=NS
)
_sc_params = pltpu.CompilerParams(use_tc_tiling_on_sc=False)


def _fill(ref, value, nrows, width):
    vec = jnp.full((16,), value, jnp.float32)

    def row(i, carry):
        for j in range(width // 16):
            ref[i, pl.ds(j * 16, 16)] = vec
        return carry

    lax.fori_loop(0, nrows, row, 0)


def _zero_acc_slice(zbuf, acc, s):
    base = s * ZROWS
    full, rem = ZROWS // CHUNK, ZROWS % CHUNK
    for k in range(full):
        pltpu.sync_copy(zbuf, acc.at[pl.ds(base + k * CHUNK, CHUNK)])
    if rem:
        pltpu.sync_copy(
            zbuf.at[pl.ds(0, rem)], acc.at[pl.ds(base + full * CHUNK, rem)]
        )


def _copy_out(acc, out_hbm, c, s):
    base = s * ZROWS
    pltpu.sync_copy(acc.at[pl.ds(base, ZROWS)], out_hbm.at[c, pl.ds(base, ZROWS)])


def _make_degree_kernel():
    @functools.partial(
        pl.kernel,
        out_type=jax.ShapeDtypeStruct((NC, NP, DEGW), jnp.float32),
        mesh=_mesh,
        scratch_types=[
            pltpu.VMEM((CPT, CHUNK), jnp.int32),
            pltpu.VMEM((CHUNK, DEGW), jnp.float32),
            pltpu.VMEM_SHARED((NP, DEGW), jnp.float32),
        ],
        compiler_params=_sc_params,
    )
    def deg_kernel(dst_hbm, out_hbm, didx, vals, acc):
        c = lax.axis_index("c")
        s = lax.axis_index("s")
        wid = c * NS + s
        pltpu.sync_copy(dst_hbm.at[pl.ds(wid * CPT, CPT)], didx)
        _fill(vals, 0.0, CHUNK, DEGW)
        _zero_acc_slice(vals, acc, s)
        _fill(vals, 1.0, CHUNK, DEGW)
        plsc.subcore_barrier()

        def step(t, carry):
            pltpu.sync_copy(vals, acc.at[didx.at[t]], add=True)
            return carry

        lax.fori_loop(0, CPT, step, 0)
        plsc.subcore_barrier()
        _copy_out(acc, out_hbm, c, s)

    return deg_kernel


GS = 4


def _make_scatter_kernel(width):
    sems = [pltpu.SemaphoreType.DMA] * (2 * GS)

    @functools.partial(
        pl.kernel,
        out_type=jax.ShapeDtypeStruct((NC, NP, width), jnp.float32),
        mesh=_mesh,
        scratch_types=[
            pltpu.VMEM((CPT, CHUNK), jnp.int32),
            pltpu.VMEM((CPT, CHUNK), jnp.int32),
            pltpu.VMEM((GS, CHUNK, width), jnp.float32),
            pltpu.VMEM_SHARED((NP, width), jnp.float32),
        ]
        + sems,
        compiler_params=_sc_params,
    )
    def scat_kernel(g_hbm, src_hbm, dst_hbm, out_hbm, sidx, didx, rows, acc, *sem):
        gsem, ssem = sem[:GS], sem[GS:]
        c = lax.axis_index("c")
        s = lax.axis_index("s")
        wid = c * NS + s
        pltpu.sync_copy(src_hbm.at[pl.ds(wid * CPT, CPT)], sidx)
        pltpu.sync_copy(dst_hbm.at[pl.ds(wid * CPT, CPT)], didx)
        _fill(rows.at[0], 0.0, CHUNK, width)
        _zero_acc_slice(rows.at[0], acc, s)
        plsc.subcore_barrier()

        def gather(t, b):
            pltpu.async_copy(g_hbm.at[sidx.at[t]], rows.at[b], gsem[b])

        def gather_wait(t, b):
            pltpu.make_async_copy(g_hbm.at[sidx.at[t]], rows.at[b], gsem[b]).wait()

        def scat(t, b):
            pltpu.async_copy(rows.at[b], acc.at[didx.at[t]], ssem[b], add=True)

        def scat_wait(t, b):
            pltpu.make_async_copy(rows.at[b], acc.at[didx.at[t]], ssem[b]).wait()

        for b in range(GS - 1):
            gather(b, b)

        def outer(o, carry):
            for b in range(GS):
                t = o * GS + b
                gather_wait(t, b)
                scat(t, b)
                nb = (b + GS - 1) % GS
                tn = t + GS - 1

                @pl.when(jnp.logical_and(tn < CPT, t > 0))
                def _():
                    scat_wait(t, nb)
                    gather(tn, nb)

                if b == 0:

                    @pl.when(t == 0)
                    def _():
                        gather(GS - 1, GS - 1)

            return carry

        lax.fori_loop(0, CPT // GS, outer, 0)
        for b in range(GS):
            scat_wait(0, b)
        plsc.subcore_barrier()
        _copy_out(acc, out_hbm, c, s)

    return scat_kernel


_degree_kernel = _make_degree_kernel()
_scatter_h = _make_scatter_kernel(H)
_scatter_c = _make_scatter_kernel(CP)

BN = 2000


def _dinv_block(da_ref, db_ref):
    deg = da_ref[:, 0:1] + db_ref[:, 0:1] + 1.0
    return lax.rsqrt(deg)


def _tc1_body(x_ref, w1_ref, da_ref, db_ref, g1_ref):
    dinv = _dinv_block(da_ref, db_ref)
    h = jnp.dot(x_ref[...], w1_ref[...], preferred_element_type=jnp.float32)
    g1_ref[...] = h * dinv


def _tc2_body(g1_ref, sa_ref, sb_ref, da_ref, db_ref, b1_ref, w2_ref, g2_ref):
    dinv = _dinv_block(da_ref, db_ref)
    h1 = dinv * (sa_ref[...] + sb_ref[...] + g1_ref[...]) + b1_ref[...]
    h1 = jnp.maximum(h1, 0.0)
    h2 = jnp.dot(h1, w2_ref[...], preferred_element_type=jnp.float32)
    g2_ref[...] = h2 * dinv


def _tc3_body(g2_ref, sa_ref, sb_ref, da_ref, db_ref, b2_ref, out_ref):
    dinv = _dinv_block(da_ref, db_ref)
    res = dinv * (sa_ref[...] + sb_ref[...] + g2_ref[...])
    out_ref[...] = res[:, :C] + b2_ref[...]


def _row_spec(w):
    return pl.BlockSpec((BN, w), lambda i: (i, 0))


def _full_spec(r, w):
    return pl.BlockSpec((r, w), lambda i: (0, 0))


@jax.jit
def kernel(x, edge_index, W1, b1, W2, b2):
    src = edge_index[0]
    dst = edge_index[1]
    pad = NCHP * CHUNK - E
    srcp = jnp.concatenate([src, jnp.zeros((pad,), jnp.int32)]).reshape(NCHP, CHUNK)
    dstp = jnp.concatenate([dst, jnp.full((pad,), N, jnp.int32)]).reshape(NCHP, CHUNK)

    deg2 = _degree_kernel(dstp)
    degA, degB = deg2[0], deg2[1]

    g1 = pl.pallas_call(
        _tc1_body,
        grid=(N // BN,),
        in_specs=[
            _row_spec(D),
            _full_spec(D, H),
            _row_spec(DEGW),
            _row_spec(DEGW),
        ],
        out_specs=_row_spec(H),
        out_shape=jax.ShapeDtypeStruct((N, H), jnp.float32),
    )(x, W1, degA, degB)

    s1 = _scatter_h(g1, srcp, dstp)

    W2p = jnp.concatenate([W2, jnp.zeros((H, CP - C), jnp.float32)], axis=1)
    g2 = pl.pallas_call(
        _tc2_body,
        grid=(N // BN,),
        in_specs=[
            _row_spec(H),
            _row_spec(H),
            _row_spec(H),
            _row_spec(DEGW),
            _row_spec(DEGW),
            _full_spec(1, H),
            _full_spec(H, CP),
        ],
        out_specs=_row_spec(CP),
        out_shape=jax.ShapeDtypeStruct((N, CP), jnp.float32),
    )(g1, s1[0], s1[1], degA, degB, b1[None, :], W2p)

    s2 = _scatter_c(g2, srcp, dstp)

    logits = pl.pallas_call(
        _tc3_body,
        grid=(N // BN,),
        in_specs=[
            _row_spec(CP),
            _row_spec(CP),
            _row_spec(CP),
            _row_spec(DEGW),
            _row_spec(DEGW),
            _full_spec(1, C),
        ],
        out_specs=_row_spec(C),
        out_shape=jax.ShapeDtypeStruct((N, C), jnp.float32),
    )(g2, s2[0], s2[1], degA, degB, b2[None, :])

    return logits
